# baseline (speedup 1.0000x reference)
import functools

import jax
import jax.numpy as jnp
import numpy as np
from jax import lax
from jax.experimental import pallas as pl
from jax.experimental.pallas import tpu as pltpu
from jax.experimental.pallas import tpu_sc as plsc


def _fps(pts, npoint, interpret=False):
    B, _, N = pts.shape

    U = 4

    def kern(pts_ref, ctr_ref, dists_ref, far_ref):
        i = pl.program_id(0)

        @pl.when(i == 0)
        def _init():
            dists_ref[...] = jnp.full((B, N), 1e10, dtype=jnp.float32)
            far_ref[...] = jnp.zeros((B, 1), jnp.int32)

        x = pts_ref[:, 0, :]
        y = pts_ref[:, 1, :]
        z = pts_ref[:, 2, :]
        iota = lax.broadcasted_iota(jnp.int32, (B, N), 1)
        far = far_ref[...]
        dists = dists_ref[...]
        cols = []
        for _ in range(U):
            oh = iota == far
            cx = jnp.sum(jnp.where(oh, x, 0.0), axis=1, keepdims=True)
            cy = jnp.sum(jnp.where(oh, y, 0.0), axis=1, keepdims=True)
            cz = jnp.sum(jnp.where(oh, z, 0.0), axis=1, keepdims=True)
            cols.append(jnp.concatenate([cx, cy, cz], axis=1))
            dx = x - cx
            dy = y - cy
            dz = z - cz
            d = (dx * dx + dy * dy) + dz * dz
            dists = jnp.minimum(dists, d)
            far = jnp.argmax(dists, axis=1).astype(jnp.int32)[:, None]
        dists_ref[...] = dists
        far_ref[...] = far
        ctr_ref[...] = jnp.stack(cols).reshape(U, B, 3, 1)

    out = pl.pallas_call(
        kern,
        grid=(npoint // U,),
        in_specs=[pl.BlockSpec((B, 3, N), lambda i: (0, 0, 0))],
        out_specs=pl.BlockSpec((U, B, 3, 1), lambda i: (i, 0, 0, 0)),
        out_shape=jax.ShapeDtypeStruct((npoint, B, 3, 1), jnp.float32),
        scratch_shapes=[
            pltpu.VMEM((B, N), jnp.float32),
            pltpu.VMEM((B, 1), jnp.int32),
        ],
        interpret=interpret,
    )(pts)
    return out.reshape(npoint, B, 3).transpose(1, 2, 0)


def _bq_scan(xs, ys, zs, ibufa, ibuf, cx, cy, cz, R2, NCH, N, iota16):
    zero16 = jnp.zeros((16,), jnp.int32)

    def group_body(g, p):
        cnts = zero16
        for j in range(16):
            c = g * 16 + j
            xv = xs[pl.ds(c * 16, 16)]
            yv = ys[pl.ds(c * 16, 16)]
            zv = zs[pl.ds(c * 16, 16)]
            dx = xv - cx
            dy = yv - cy
            dz = zv - cz
            d = (dx * dx + dy * dy) + dz * dz
            m = d <= R2
            incl = plsc.cumsum(m.astype(jnp.int32))
            tgt = jnp.where(m, c * 16 + (incl - 1), N + iota16)
            plsc.store_scatter(ibufa, [tgt], iota16 + c * 16)
            cnts = jnp.where(iota16 == j, incl[15], cnts)
        offs = plsc.cumsum(cnts)
        excl = offs - cnts
        for j in range(16):
            c = g * 16 + j
            iv = ibufa[pl.ds(c * 16, 16)]
            tgt = jnp.where(iota16 < cnts[j], p + excl[j] + iota16,
                            N + iota16)
            plsc.store_scatter(ibuf, [tgt], iv)
        return p + offs[15]

    return lax.fori_loop(0, NCH // 16, group_body, jnp.int32(0))


def _bq_gather_xyz(soa, ctrs, radius):
    B, _, N = soa.shape
    S = ctrs.shape[2]
    KN = 64
    WPS = 32 // B
    SCH = S // WPS
    NCH = N // 16
    R2 = np.float32(np.float64(radius) * np.float64(radius))
    mesh = plsc.VectorSubcoreMesh(core_axis_name="c", subcore_axis_name="s")

    @functools.partial(
        pl.kernel,
        out_type=jax.ShapeDtypeStruct((B * S * KN * 3,), jnp.float32),
        mesh=mesh,
        scratch_types=[
            pltpu.VMEM((N,), jnp.float32),
            pltpu.VMEM((N,), jnp.float32),
            pltpu.VMEM((N,), jnp.float32),
            pltpu.VMEM((SCH + 16,), jnp.float32),
            pltpu.VMEM((SCH + 16,), jnp.float32),
            pltpu.VMEM((SCH + 16,), jnp.float32),
            pltpu.VMEM((N + 32,), jnp.int32),
            pltpu.VMEM((N + 32,), jnp.int32),
            pltpu.VMEM((SCH * KN * 3,), jnp.float32),
        ],
        compiler_params=pltpu.CompilerParams(needs_layout_passes=False),
    )
    def bq(soa_hbm, ctr_hbm, out_hbm,
           xs, ys, zs, cxv, cyv, czv, ibufa, ibuf, outv):
        b = lax.axis_index("s")
        h = lax.axis_index("c")
        pltpu.sync_copy(soa_hbm.at[pl.ds((b * 3 + 0) * N, N)], xs)
        pltpu.sync_copy(soa_hbm.at[pl.ds((b * 3 + 1) * N, N)], ys)
        pltpu.sync_copy(soa_hbm.at[pl.ds((b * 3 + 2) * N, N)], zs)
        pltpu.sync_copy(ctr_hbm.at[pl.ds((b * 3 + 0) * S + h * SCH, SCH)],
                        cxv.at[pl.ds(0, SCH)])
        pltpu.sync_copy(ctr_hbm.at[pl.ds((b * 3 + 1) * S + h * SCH, SCH)],
                        cyv.at[pl.ds(0, SCH)])
        pltpu.sync_copy(ctr_hbm.at[pl.ds((b * 3 + 2) * S + h * SCH, SCH)],
                        czv.at[pl.ds(0, SCH)])
        iota16 = lax.iota(jnp.int32, 16)

        def center_body(sl, _):
            cx = cxv[pl.ds(sl, 16)][0]
            cy = cyv[pl.ds(sl, 16)][0]
            cz = czv[pl.ds(sl, 16)][0]
            p = _bq_scan(xs, ys, zs, ibufa, ibuf, cx, cy, cz, R2, NCH, N, iota16)
            count = jnp.minimum(p, KN)
            first = ibuf[pl.ds(0, 16)][0]
            for j4 in range(KN // 16):
                iv = ibuf[pl.ds(j4 * 16, 16)]
                pos = iota16 + j4 * 16
                iv = jnp.where(pos < count, iv, first)
                tgt = sl * KN + pos
                plsc.store_scatter(outv, [tgt],
                                   plsc.load_gather(xs, [iv]) - cx)
                plsc.store_scatter(outv, [tgt + SCH * KN],
                                   plsc.load_gather(ys, [iv]) - cy)
                plsc.store_scatter(outv, [tgt + 2 * SCH * KN],
                                   plsc.load_gather(zs, [iv]) - cz)
            return 0

        lax.fori_loop(0, SCH, center_body, 0)
        base = (b * S + h * SCH) * KN
        for pidx in range(3):
            pltpu.sync_copy(
                outv.at[pl.ds(pidx * SCH * KN, SCH * KN)],
                out_hbm.at[pl.ds(pidx * B * S * KN + base, SCH * KN)])

    return bq(soa.reshape(B * 3 * N), ctrs.reshape(B * 3 * S))


def _bq_gather_feat(soa, ctrs, tab, radius):
    B, _, N = soa.shape
    S = ctrs.shape[2]
    CR = tab.shape[1]
    KN = 64
    WPS = 32 // B
    SCH = S // WPS
    NCH = N // 16
    R2 = np.float32(np.float64(radius) * np.float64(radius))
    mesh = plsc.VectorSubcoreMesh(core_axis_name="c", subcore_axis_name="s")

    @functools.partial(
        pl.kernel,
        out_type=jax.ShapeDtypeStruct((B * S * KN, CR), jnp.float32),
        mesh=mesh,
        scratch_types=[
            pltpu.VMEM((N,), jnp.float32),
            pltpu.VMEM((N,), jnp.float32),
            pltpu.VMEM((N,), jnp.float32),
            pltpu.VMEM((SCH + 16,), jnp.float32),
            pltpu.VMEM((SCH + 16,), jnp.float32),
            pltpu.VMEM((SCH + 16,), jnp.float32),
            pltpu.VMEM((N + 32,), jnp.int32),
            pltpu.VMEM((N + 32,), jnp.int32),
            pltpu.VMEM((KN,), jnp.int32),
            pltpu.VMEM((KN, CR), jnp.float32),
            pltpu.SemaphoreType.DMA,
            pltpu.SemaphoreType.DMA,
        ],
        compiler_params=pltpu.CompilerParams(needs_layout_passes=False),
    )
    def bq(soa_hbm, ctr_hbm, tab_hbm, out_hbm,
           xs, ys, zs, cxv, cyv, czv, ibufa, ibuf, idxb, rowb, gsem, osem):
        b = lax.axis_index("s")
        h = lax.axis_index("c")
        pltpu.sync_copy(soa_hbm.at[pl.ds((b * 3 + 0) * N, N)], xs)
        pltpu.sync_copy(soa_hbm.at[pl.ds((b * 3 + 1) * N, N)], ys)
        pltpu.sync_copy(soa_hbm.at[pl.ds((b * 3 + 2) * N, N)], zs)
        pltpu.sync_copy(ctr_hbm.at[pl.ds((b * 3 + 0) * S + h * SCH, SCH)],
                        cxv.at[pl.ds(0, SCH)])
        pltpu.sync_copy(ctr_hbm.at[pl.ds((b * 3 + 1) * S + h * SCH, SCH)],
                        cyv.at[pl.ds(0, SCH)])
        pltpu.sync_copy(ctr_hbm.at[pl.ds((b * 3 + 2) * S + h * SCH, SCH)],
                        czv.at[pl.ds(0, SCH)])
        iota16 = lax.iota(jnp.int32, 16)

        def center_body(sl, _):
            cx = cxv[pl.ds(sl, 16)][0]
            cy = cyv[pl.ds(sl, 16)][0]
            cz = czv[pl.ds(sl, 16)][0]
            p = _bq_scan(xs, ys, zs, ibufa, ibuf, cx, cy, cz, R2, NCH, N, iota16)
            count = jnp.minimum(p, KN)
            first = ibuf[pl.ds(0, 16)][0]
            for j4 in range(KN // 16):
                iv = ibuf[pl.ds(j4 * 16, 16)]
                pos = iota16 + j4 * 16
                iv = jnp.where(pos < count, iv, first)
                idxb[pl.ds(j4 * 16, 16)] = iv + b * N

            pltpu.async_copy(tab_hbm.at[idxb], rowb, gsem).wait()
            row0 = (b * S + h * SCH + sl) * KN
            pltpu.async_copy(rowb, out_hbm.at[pl.ds(row0, KN)], osem).wait()
            return 0

        lax.fori_loop(0, SCH, center_body, 0)

    return bq(soa.reshape(B * 3 * N), ctrs.reshape(B * 3 * S), tab)


def _mlp_max_t(gT, wbs, group=64, tm=8192, interpret=False):
    Cin, R = gT.shape
    G = tm // group
    nlayer = len(wbs)
    Cout = wbs[-1][0].shape[1]

    def kern(g_ref, *refs):
        o_ref = refs[-1]
        h = g_ref[...]
        for i in range(nlayer):
            wT = refs[2 * i][...]
            bc = refs[2 * i + 1][...]
            h = jnp.maximum(
                jnp.dot(wT, h, preferred_element_type=jnp.float32) + bc, 0.0)
        t = h.T
        o_ref[...] = jnp.max(t.reshape(G, group, Cout), axis=1)

    in_specs = [pl.BlockSpec((Cin, tm), lambda i: (0, i))]
    args = [gT]
    for W, bvec in wbs:
        in_specs.append(pl.BlockSpec(W.shape[::-1], lambda i: (0, 0)))
        in_specs.append(pl.BlockSpec((W.shape[1], 1), lambda i: (0, 0)))
        args.append(W.T)
        args.append(bvec.reshape(-1, 1))

    return pl.pallas_call(
        kern,
        grid=(R // tm,),
        in_specs=in_specs,
        out_specs=pl.BlockSpec((G, Cout), lambda i: (i, 0)),
        out_shape=jax.ShapeDtypeStruct((R // group, Cout), jnp.float32),
        interpret=interpret,
    )(*args)


def _proj2(ctr1_soa, f1, ctr2_soa, W1, b1, interpret=False):
    _, R1 = ctr1_soa.shape
    _, R2 = ctr2_soa.shape
    C = W1.shape[1]
    W1xT = W1[0:3].T
    W1f = W1[3:]

    def kern(c1_ref, f1_ref, c2_ref, wxT_ref, wf_ref, b_ref, a_ref, d_ref):
        wxT = wxT_ref[...]
        a_ref[...] = (
            jnp.dot(wxT, c1_ref[...], preferred_element_type=jnp.float32).T
            + jnp.dot(f1_ref[...], wf_ref[...],
                      preferred_element_type=jnp.float32)
            + b_ref[...])
        d_ref[...] = jnp.dot(
            wxT, c2_ref[...], preferred_element_type=jnp.float32).T

    return pl.pallas_call(
        kern,
        out_shape=(jax.ShapeDtypeStruct((R1, C), jnp.float32),
                   jax.ShapeDtypeStruct((R2, C), jnp.float32)),
        interpret=interpret,
    )(ctr1_soa, f1, ctr2_soa, W1xT, W1f, b1.reshape(1, -1))


def _mlp_max_sub(gA, D, wbs, group=64, tm=8192, interpret=False):
    R, C = gA.shape
    G = tm // group
    Cout = wbs[-1][0].shape[1]
    nlayer = len(wbs)

    def kern(g_ref, d_ref, *refs):
        o_ref = refs[-1]
        h = g_ref[...]
        d = d_ref[...]
        h = jnp.maximum(
            (h.reshape(G, group, C) - d[:, None, :]).reshape(tm, C), 0.0)
        for i in range(nlayer):
            W = refs[2 * i][...]
            bvec = refs[2 * i + 1][...]
            h = jnp.maximum(
                jnp.dot(h, W, preferred_element_type=jnp.float32) + bvec, 0.0)
        o_ref[...] = jnp.max(h.reshape(G, group, h.shape[-1]), axis=1)

    in_specs = [
        pl.BlockSpec((tm, C), lambda i: (i, 0)),
        pl.BlockSpec((G, C), lambda i: (i, 0)),
    ]
    args = [gA, D]
    for W, bvec in wbs:
        in_specs.append(pl.BlockSpec(W.shape, lambda i: (0, 0)))
        in_specs.append(pl.BlockSpec((1, W.shape[1]), lambda i: (0, 0)))
        args.append(W)
        args.append(bvec.reshape(1, -1))

    return pl.pallas_call(
        kern,
        grid=(R // tm,),
        in_specs=in_specs,
        out_specs=pl.BlockSpec((G, Cout), lambda i: (i, 0)),
        out_shape=jax.ShapeDtypeStruct((R // group, Cout), jnp.float32),
        interpret=interpret,
    )(*args)


def _mlp_max(g, cpad, wbs, group=64, tm=8192, interpret=False):
    R, Cin = g.shape
    G = tm // group
    grid = R // tm
    Cout = wbs[-1][0].shape[1]
    nlayer = len(wbs)

    def kern(g_ref, c_ref, *refs):
        o_ref = refs[-1]
        h = g_ref[...]
        ctr = c_ref[...]
        h = (h.reshape(G, group, Cin) - ctr[:, None, :]).reshape(tm, Cin)
        for i in range(nlayer):
            W = refs[2 * i][...]
            bvec = refs[2 * i + 1][...]
            h = jnp.maximum(
                jnp.dot(h, W, preferred_element_type=jnp.float32) + bvec, 0.0)
        o_ref[...] = jnp.max(h.reshape(G, group, h.shape[-1]), axis=1)

    in_specs = [
        pl.BlockSpec((tm, Cin), lambda i: (i, 0)),
        pl.BlockSpec((G, Cin), lambda i: (i, 0)),
    ]
    args = [g, cpad]
    for W, bvec in wbs:
        in_specs.append(pl.BlockSpec(W.shape, lambda i: (0, 0)))
        in_specs.append(pl.BlockSpec((1, W.shape[1]), lambda i: (0, 0)))
        args.append(W)
        args.append(bvec.reshape(1, -1))

    return pl.pallas_call(
        kern,
        grid=(grid,),
        in_specs=in_specs,
        out_specs=pl.BlockSpec((G, Cout), lambda i: (i, 0)),
        out_shape=jax.ShapeDtypeStruct((R // group, Cout), jnp.float32),
        interpret=interpret,
    )(*args)


def _head(rows, B, wbs, interpret=False):
    R, Cin = rows.shape
    S = R // B
    nlayer = len(wbs)

    def kern(r_ref, *refs):
        o_ref = refs[-1]
        h = r_ref[...]
        for i in range(3):
            W = refs[2 * i][...]
            bvec = refs[2 * i + 1][...]
            h = jnp.maximum(
                jnp.dot(h, W, preferred_element_type=jnp.float32) + bvec, 0.0)
        x = jnp.max(h.reshape(B, S, h.shape[-1]), axis=1)
        for i in range(3, nlayer - 1):
            W = refs[2 * i][...]
            bvec = refs[2 * i + 1][...]
            x = jnp.maximum(
                jnp.dot(x, W, preferred_element_type=jnp.float32) + bvec, 0.0)
        W = refs[2 * (nlayer - 1)][...]
        bvec = refs[2 * (nlayer - 1) + 1][...]
        o_ref[...] = jnp.dot(x, W, preferred_element_type=jnp.float32) + bvec

    args = [rows]
    for W, bvec in wbs:
        args.append(W)
        args.append(bvec.reshape(1, -1))

    return pl.pallas_call(
        kern,
        out_shape=jax.ShapeDtypeStruct((B, wbs[-1][0].shape[1]), jnp.float32),
        interpret=interpret,
    )(*args)


def _pad_rows(W, rows_to):
    return jnp.concatenate(
        [W, jnp.zeros((rows_to - W.shape[0], W.shape[1]), W.dtype)], axis=0)


def kernel(points, params):
    sa1, sa2, sa3, fc, lin = params
    B, _, N1 = points.shape
    S1, S2, KN = 512, 128, 64

    ctr1 = _fps(points, S1)
    g1 = _bq_gather_xyz(points, ctr1, 0.2).reshape(3, B * S1 * KN)
    ctr1_rows = ctr1.transpose(0, 2, 1).reshape(B * S1, 3)
    f1 = _mlp_max_t(g1, list(sa1), group=KN)

    ctr2 = _fps(ctr1, S2)
    ctr1_soa = ctr1.transpose(1, 0, 2).reshape(3, B * S1)
    ctr2_soa = ctr2.transpose(1, 0, 2).reshape(3, B * S2)
    A2, D2 = _proj2(ctr1_soa, f1, ctr2_soa, sa2[0][0], sa2[0][1])
    gA = _bq_gather_feat(ctr1, ctr2, A2, 0.4)
    ctr2_rows = ctr2.transpose(0, 2, 1).reshape(B * S2, 3)
    f2 = _mlp_max_sub(gA, D2, list(sa2[1:]), group=KN)

    rows3 = jnp.concatenate([ctr2_rows, f2], axis=1)
    wbs3 = list(sa3) + list(fc) + [lin]
    return _head(rows3, B, wbs3)


# speedup vs baseline: 1.0740x; 1.0740x over previous
import functools

import jax
import jax.numpy as jnp
import numpy as np
from jax import lax
from jax.experimental import pallas as pl
from jax.experimental.pallas import tpu as pltpu
from jax.experimental.pallas import tpu_sc as plsc


def _fps(pts, npoint, interpret=False):
    B, _, N = pts.shape

    U = 4

    def kern(pts_ref, ctr_ref, dists_ref, far_ref):
        i = pl.program_id(0)

        @pl.when(i == 0)
        def _init():
            dists_ref[...] = jnp.full((B, N), 1e10, dtype=jnp.float32)
            far_ref[...] = jnp.zeros((B, 1), jnp.int32)

        x = pts_ref[:, 0, :]
        y = pts_ref[:, 1, :]
        z = pts_ref[:, 2, :]
        iota = lax.broadcasted_iota(jnp.int32, (B, N), 1)
        far = far_ref[...]
        dists = dists_ref[...]
        cols = []
        for _ in range(U):
            oh = iota == far
            cx = jnp.sum(jnp.where(oh, x, 0.0), axis=1, keepdims=True)
            cy = jnp.sum(jnp.where(oh, y, 0.0), axis=1, keepdims=True)
            cz = jnp.sum(jnp.where(oh, z, 0.0), axis=1, keepdims=True)
            cols.append(jnp.concatenate([cx, cy, cz], axis=1))
            dx = x - cx
            dy = y - cy
            dz = z - cz
            d = (dx * dx + dy * dy) + dz * dz
            dists = jnp.minimum(dists, d)
            far = jnp.argmax(dists, axis=1).astype(jnp.int32)[:, None]
        dists_ref[...] = dists
        far_ref[...] = far
        ctr_ref[...] = jnp.stack(cols).reshape(U, B, 3, 1)

    out = pl.pallas_call(
        kern,
        grid=(npoint // U,),
        in_specs=[pl.BlockSpec((B, 3, N), lambda i: (0, 0, 0))],
        out_specs=pl.BlockSpec((U, B, 3, 1), lambda i: (i, 0, 0, 0)),
        out_shape=jax.ShapeDtypeStruct((npoint, B, 3, 1), jnp.float32),
        scratch_shapes=[
            pltpu.VMEM((B, N), jnp.float32),
            pltpu.VMEM((B, 1), jnp.int32),
        ],
        interpret=interpret,
    )(pts)
    return out.reshape(npoint, B, 3).transpose(1, 2, 0)


def _bq_scan(xs, ys, zs, ibufa, ibuf, cx, cy, cz, R2, NCH, N, iota16):
    del ibufa

    def group_body(g, p):
        for j in range(4):
            c = g * 4 + j
            xv = xs[pl.ds(c * 16, 16)]
            yv = ys[pl.ds(c * 16, 16)]
            zv = zs[pl.ds(c * 16, 16)]
            dx = xv - cx
            dy = yv - cy
            dz = zv - cz
            d = (dx * dx + dy * dy) + dz * dz
            m = d <= R2
            incl = plsc.cumsum(m.astype(jnp.int32))
            tgt = jnp.where(m, p + (incl - 1), N + iota16)
            plsc.store_scatter(ibuf, [tgt], iota16 + c * 16)
            cnt = plsc.all_reduce_population_count(m)
            p = p + (cnt[0] if cnt.ndim else cnt)
        return p

    return lax.fori_loop(0, NCH // 4, group_body, jnp.int32(0))


def _bq_gather_xyz(soa, ctrs, radius):
    B, _, N = soa.shape
    S = ctrs.shape[2]
    KN = 64
    WPS = 32 // B
    SCH = S // WPS
    NCH = N // 16
    R2 = np.float32(np.float64(radius) * np.float64(radius))
    mesh = plsc.VectorSubcoreMesh(core_axis_name="c", subcore_axis_name="s")

    @functools.partial(
        pl.kernel,
        out_type=jax.ShapeDtypeStruct((B * S * KN * 3,), jnp.float32),
        mesh=mesh,
        scratch_types=[
            pltpu.VMEM((N,), jnp.float32),
            pltpu.VMEM((N,), jnp.float32),
            pltpu.VMEM((N,), jnp.float32),
            pltpu.VMEM((SCH + 16,), jnp.float32),
            pltpu.VMEM((SCH + 16,), jnp.float32),
            pltpu.VMEM((SCH + 16,), jnp.float32),
            pltpu.VMEM((N + 32,), jnp.int32),
            pltpu.VMEM((N + 32,), jnp.int32),
            pltpu.VMEM((SCH * KN * 3,), jnp.float32),
        ],
        compiler_params=pltpu.CompilerParams(needs_layout_passes=False),
    )
    def bq(soa_hbm, ctr_hbm, out_hbm,
           xs, ys, zs, cxv, cyv, czv, ibufa, ibuf, outv):
        b = lax.axis_index("s")
        h = lax.axis_index("c")
        pltpu.sync_copy(soa_hbm.at[pl.ds((b * 3 + 0) * N, N)], xs)
        pltpu.sync_copy(soa_hbm.at[pl.ds((b * 3 + 1) * N, N)], ys)
        pltpu.sync_copy(soa_hbm.at[pl.ds((b * 3 + 2) * N, N)], zs)
        pltpu.sync_copy(ctr_hbm.at[pl.ds((b * 3 + 0) * S + h * SCH, SCH)],
                        cxv.at[pl.ds(0, SCH)])
        pltpu.sync_copy(ctr_hbm.at[pl.ds((b * 3 + 1) * S + h * SCH, SCH)],
                        cyv.at[pl.ds(0, SCH)])
        pltpu.sync_copy(ctr_hbm.at[pl.ds((b * 3 + 2) * S + h * SCH, SCH)],
                        czv.at[pl.ds(0, SCH)])
        iota16 = lax.iota(jnp.int32, 16)

        def center_body(sl, _):
            cx = cxv[pl.ds(sl, 16)][0]
            cy = cyv[pl.ds(sl, 16)][0]
            cz = czv[pl.ds(sl, 16)][0]
            p = _bq_scan(xs, ys, zs, ibufa, ibuf, cx, cy, cz, R2, NCH, N, iota16)
            count = jnp.minimum(p, KN)
            first = ibuf[pl.ds(0, 16)][0]
            for j4 in range(KN // 16):
                iv = ibuf[pl.ds(j4 * 16, 16)]
                pos = iota16 + j4 * 16
                iv = jnp.where(pos < count, iv, first)
                tgt = sl * KN + pos
                plsc.store_scatter(outv, [tgt],
                                   plsc.load_gather(xs, [iv]) - cx)
                plsc.store_scatter(outv, [tgt + SCH * KN],
                                   plsc.load_gather(ys, [iv]) - cy)
                plsc.store_scatter(outv, [tgt + 2 * SCH * KN],
                                   plsc.load_gather(zs, [iv]) - cz)
            return 0

        lax.fori_loop(0, SCH, center_body, 0)
        base = (b * S + h * SCH) * KN
        for pidx in range(3):
            pltpu.sync_copy(
                outv.at[pl.ds(pidx * SCH * KN, SCH * KN)],
                out_hbm.at[pl.ds(pidx * B * S * KN + base, SCH * KN)])

    return bq(soa.reshape(B * 3 * N), ctrs.reshape(B * 3 * S))


def _bq_gather_feat(soa, ctrs, tab, radius):
    B, _, N = soa.shape
    S = ctrs.shape[2]
    CR = tab.shape[1]
    KN = 64
    WPS = 32 // B
    SCH = S // WPS
    NCH = N // 16
    R2 = np.float32(np.float64(radius) * np.float64(radius))
    mesh = plsc.VectorSubcoreMesh(core_axis_name="c", subcore_axis_name="s")

    @functools.partial(
        pl.kernel,
        out_type=jax.ShapeDtypeStruct((B * S * KN, CR), jnp.float32),
        mesh=mesh,
        scratch_types=[
            pltpu.VMEM((N,), jnp.float32),
            pltpu.VMEM((N,), jnp.float32),
            pltpu.VMEM((N,), jnp.float32),
            pltpu.VMEM((SCH + 16,), jnp.float32),
            pltpu.VMEM((SCH + 16,), jnp.float32),
            pltpu.VMEM((SCH + 16,), jnp.float32),
            pltpu.VMEM((N + 32,), jnp.int32),
            pltpu.VMEM((N + 32,), jnp.int32),
            pltpu.VMEM((KN,), jnp.int32),
            pltpu.VMEM((KN, CR), jnp.float32),
            pltpu.SemaphoreType.DMA,
            pltpu.SemaphoreType.DMA,
        ],
        compiler_params=pltpu.CompilerParams(needs_layout_passes=False),
    )
    def bq(soa_hbm, ctr_hbm, tab_hbm, out_hbm,
           xs, ys, zs, cxv, cyv, czv, ibufa, ibuf, idxb, rowb, gsem, osem):
        b = lax.axis_index("s")
        h = lax.axis_index("c")
        pltpu.sync_copy(soa_hbm.at[pl.ds((b * 3 + 0) * N, N)], xs)
        pltpu.sync_copy(soa_hbm.at[pl.ds((b * 3 + 1) * N, N)], ys)
        pltpu.sync_copy(soa_hbm.at[pl.ds((b * 3 + 2) * N, N)], zs)
        pltpu.sync_copy(ctr_hbm.at[pl.ds((b * 3 + 0) * S + h * SCH, SCH)],
                        cxv.at[pl.ds(0, SCH)])
        pltpu.sync_copy(ctr_hbm.at[pl.ds((b * 3 + 1) * S + h * SCH, SCH)],
                        cyv.at[pl.ds(0, SCH)])
        pltpu.sync_copy(ctr_hbm.at[pl.ds((b * 3 + 2) * S + h * SCH, SCH)],
                        czv.at[pl.ds(0, SCH)])
        iota16 = lax.iota(jnp.int32, 16)

        def center_body(sl, _):
            cx = cxv[pl.ds(sl, 16)][0]
            cy = cyv[pl.ds(sl, 16)][0]
            cz = czv[pl.ds(sl, 16)][0]
            p = _bq_scan(xs, ys, zs, ibufa, ibuf, cx, cy, cz, R2, NCH, N, iota16)
            count = jnp.minimum(p, KN)
            first = ibuf[pl.ds(0, 16)][0]
            for j4 in range(KN // 16):
                iv = ibuf[pl.ds(j4 * 16, 16)]
                pos = iota16 + j4 * 16
                iv = jnp.where(pos < count, iv, first)
                idxb[pl.ds(j4 * 16, 16)] = iv + b * N

            pltpu.async_copy(tab_hbm.at[idxb], rowb, gsem).wait()
            row0 = (b * S + h * SCH + sl) * KN
            pltpu.async_copy(rowb, out_hbm.at[pl.ds(row0, KN)], osem).wait()
            return 0

        lax.fori_loop(0, SCH, center_body, 0)

    return bq(soa.reshape(B * 3 * N), ctrs.reshape(B * 3 * S), tab)


def _mlp_max_t(gT, wbs, group=64, tm=8192, interpret=False):
    Cin, R = gT.shape
    G = tm // group
    nlayer = len(wbs)
    Cout = wbs[-1][0].shape[1]

    def kern(g_ref, *refs):
        o_ref = refs[-1]
        h = g_ref[...]
        for i in range(nlayer):
            wT = refs[2 * i][...]
            bc = refs[2 * i + 1][...]
            h = jnp.maximum(
                jnp.dot(wT, h, preferred_element_type=jnp.float32) + bc, 0.0)
        t = h.T
        o_ref[...] = jnp.max(t.reshape(G, group, Cout), axis=1)

    in_specs = [pl.BlockSpec((Cin, tm), lambda i: (0, i))]
    args = [gT]
    for W, bvec in wbs:
        in_specs.append(pl.BlockSpec(W.shape[::-1], lambda i: (0, 0)))
        in_specs.append(pl.BlockSpec((W.shape[1], 1), lambda i: (0, 0)))
        args.append(W.T)
        args.append(bvec.reshape(-1, 1))

    return pl.pallas_call(
        kern,
        grid=(R // tm,),
        in_specs=in_specs,
        out_specs=pl.BlockSpec((G, Cout), lambda i: (i, 0)),
        out_shape=jax.ShapeDtypeStruct((R // group, Cout), jnp.float32),
        interpret=interpret,
    )(*args)


def _proj2(ctr1_soa, f1, ctr2_soa, W1, b1, interpret=False):
    _, R1 = ctr1_soa.shape
    _, R2 = ctr2_soa.shape
    C = W1.shape[1]
    W1xT = W1[0:3].T
    W1f = W1[3:]

    def kern(c1_ref, f1_ref, c2_ref, wxT_ref, wf_ref, b_ref, a_ref, d_ref):
        wxT = wxT_ref[...]
        a_ref[...] = (
            jnp.dot(wxT, c1_ref[...], preferred_element_type=jnp.float32).T
            + jnp.dot(f1_ref[...], wf_ref[...],
                      preferred_element_type=jnp.float32)
            + b_ref[...])
        d_ref[...] = jnp.dot(
            wxT, c2_ref[...], preferred_element_type=jnp.float32).T

    return pl.pallas_call(
        kern,
        out_shape=(jax.ShapeDtypeStruct((R1, C), jnp.float32),
                   jax.ShapeDtypeStruct((R2, C), jnp.float32)),
        interpret=interpret,
    )(ctr1_soa, f1, ctr2_soa, W1xT, W1f, b1.reshape(1, -1))


def _mlp_max_sub(gA, D, wbs, group=64, tm=8192, interpret=False):
    R, C = gA.shape
    G = tm // group
    Cout = wbs[-1][0].shape[1]
    nlayer = len(wbs)

    def kern(g_ref, d_ref, *refs):
        o_ref = refs[-1]
        h = g_ref[...]
        d = d_ref[...]
        h = jnp.maximum(
            (h.reshape(G, group, C) - d[:, None, :]).reshape(tm, C), 0.0)
        for i in range(nlayer):
            W = refs[2 * i][...]
            bvec = refs[2 * i + 1][...]
            h = jnp.maximum(
                jnp.dot(h, W, preferred_element_type=jnp.float32) + bvec, 0.0)
        o_ref[...] = jnp.max(h.reshape(G, group, h.shape[-1]), axis=1)

    in_specs = [
        pl.BlockSpec((tm, C), lambda i: (i, 0)),
        pl.BlockSpec((G, C), lambda i: (i, 0)),
    ]
    args = [gA, D]
    for W, bvec in wbs:
        in_specs.append(pl.BlockSpec(W.shape, lambda i: (0, 0)))
        in_specs.append(pl.BlockSpec((1, W.shape[1]), lambda i: (0, 0)))
        args.append(W)
        args.append(bvec.reshape(1, -1))

    return pl.pallas_call(
        kern,
        grid=(R // tm,),
        in_specs=in_specs,
        out_specs=pl.BlockSpec((G, Cout), lambda i: (i, 0)),
        out_shape=jax.ShapeDtypeStruct((R // group, Cout), jnp.float32),
        interpret=interpret,
    )(*args)


def _mlp_max(g, cpad, wbs, group=64, tm=8192, interpret=False):
    R, Cin = g.shape
    G = tm // group
    grid = R // tm
    Cout = wbs[-1][0].shape[1]
    nlayer = len(wbs)

    def kern(g_ref, c_ref, *refs):
        o_ref = refs[-1]
        h = g_ref[...]
        ctr = c_ref[...]
        h = (h.reshape(G, group, Cin) - ctr[:, None, :]).reshape(tm, Cin)
        for i in range(nlayer):
            W = refs[2 * i][...]
            bvec = refs[2 * i + 1][...]
            h = jnp.maximum(
                jnp.dot(h, W, preferred_element_type=jnp.float32) + bvec, 0.0)
        o_ref[...] = jnp.max(h.reshape(G, group, h.shape[-1]), axis=1)

    in_specs = [
        pl.BlockSpec((tm, Cin), lambda i: (i, 0)),
        pl.BlockSpec((G, Cin), lambda i: (i, 0)),
    ]
    args = [g, cpad]
    for W, bvec in wbs:
        in_specs.append(pl.BlockSpec(W.shape, lambda i: (0, 0)))
        in_specs.append(pl.BlockSpec((1, W.shape[1]), lambda i: (0, 0)))
        args.append(W)
        args.append(bvec.reshape(1, -1))

    return pl.pallas_call(
        kern,
        grid=(grid,),
        in_specs=in_specs,
        out_specs=pl.BlockSpec((G, Cout), lambda i: (i, 0)),
        out_shape=jax.ShapeDtypeStruct((R // group, Cout), jnp.float32),
        interpret=interpret,
    )(*args)


def _head(rows, B, wbs, interpret=False):
    R, Cin = rows.shape
    S = R // B
    nlayer = len(wbs)

    def kern(r_ref, *refs):
        o_ref = refs[-1]
        h = r_ref[...]
        for i in range(3):
            W = refs[2 * i][...]
            bvec = refs[2 * i + 1][...]
            h = jnp.maximum(
                jnp.dot(h, W, preferred_element_type=jnp.float32) + bvec, 0.0)
        x = jnp.max(h.reshape(B, S, h.shape[-1]), axis=1)
        for i in range(3, nlayer - 1):
            W = refs[2 * i][...]
            bvec = refs[2 * i + 1][...]
            x = jnp.maximum(
                jnp.dot(x, W, preferred_element_type=jnp.float32) + bvec, 0.0)
        W = refs[2 * (nlayer - 1)][...]
        bvec = refs[2 * (nlayer - 1) + 1][...]
        o_ref[...] = jnp.dot(x, W, preferred_element_type=jnp.float32) + bvec

    args = [rows]
    for W, bvec in wbs:
        args.append(W)
        args.append(bvec.reshape(1, -1))

    return pl.pallas_call(
        kern,
        out_shape=jax.ShapeDtypeStruct((B, wbs[-1][0].shape[1]), jnp.float32),
        interpret=interpret,
    )(*args)


def _pad_rows(W, rows_to):
    return jnp.concatenate(
        [W, jnp.zeros((rows_to - W.shape[0], W.shape[1]), W.dtype)], axis=0)


def kernel(points, params):
    sa1, sa2, sa3, fc, lin = params
    B, _, N1 = points.shape
    S1, S2, KN = 512, 128, 64

    ctr1 = _fps(points, S1)
    g1 = _bq_gather_xyz(points, ctr1, 0.2).reshape(3, B * S1 * KN)
    ctr1_rows = ctr1.transpose(0, 2, 1).reshape(B * S1, 3)
    f1 = _mlp_max_t(g1, list(sa1), group=KN)

    ctr2 = _fps(ctr1, S2)
    ctr1_soa = ctr1.transpose(1, 0, 2).reshape(3, B * S1)
    ctr2_soa = ctr2.transpose(1, 0, 2).reshape(3, B * S2)
    A2, D2 = _proj2(ctr1_soa, f1, ctr2_soa, sa2[0][0], sa2[0][1])
    gA = _bq_gather_feat(ctr1, ctr2, A2, 0.4)
    ctr2_rows = ctr2.transpose(0, 2, 1).reshape(B * S2, 3)
    f2 = _mlp_max_sub(gA, D2, list(sa2[1:]), group=KN)

    rows3 = jnp.concatenate([ctr2_rows, f2], axis=1)
    wbs3 = list(sa3) + list(fc) + [lin]
    return _head(rows3, B, wbs3)


# speedup vs baseline: 1.1786x; 1.0974x over previous
import functools

import jax
import jax.numpy as jnp
import numpy as np
from jax import lax
from jax.experimental import pallas as pl
from jax.experimental.pallas import tpu as pltpu
from jax.experimental.pallas import tpu_sc as plsc


def _fps(pts, npoint, interpret=False):
    B, _, N = pts.shape

    U = 4

    def kern(pts_ref, ctr_ref, dists_ref, far_ref):
        i = pl.program_id(0)

        @pl.when(i == 0)
        def _init():
            dists_ref[...] = jnp.full((B, N), 1e10, dtype=jnp.float32)
            far_ref[...] = jnp.zeros((B, 1), jnp.int32)

        x = pts_ref[:, 0, :]
        y = pts_ref[:, 1, :]
        z = pts_ref[:, 2, :]
        iota = lax.broadcasted_iota(jnp.int32, (B, N), 1)
        far = far_ref[...]
        dists = dists_ref[...]
        cols = []
        for _ in range(U):
            oh = iota == far
            cx = jnp.sum(jnp.where(oh, x, 0.0), axis=1, keepdims=True)
            cy = jnp.sum(jnp.where(oh, y, 0.0), axis=1, keepdims=True)
            cz = jnp.sum(jnp.where(oh, z, 0.0), axis=1, keepdims=True)
            cols.append(jnp.concatenate([cx, cy, cz], axis=1))
            dx = x - cx
            dy = y - cy
            dz = z - cz
            d = (dx * dx + dy * dy) + dz * dz
            dists = jnp.minimum(dists, d)
            far = jnp.argmax(dists, axis=1).astype(jnp.int32)[:, None]
        dists_ref[...] = dists
        far_ref[...] = far
        ctr_ref[...] = jnp.stack(cols).reshape(U, B, 3, 1)

    out = pl.pallas_call(
        kern,
        grid=(npoint // U,),
        in_specs=[pl.BlockSpec((B, 3, N), lambda i: (0, 0, 0))],
        out_specs=pl.BlockSpec((U, B, 3, 1), lambda i: (i, 0, 0, 0)),
        out_shape=jax.ShapeDtypeStruct((npoint, B, 3, 1), jnp.float32),
        scratch_shapes=[
            pltpu.VMEM((B, N), jnp.float32),
            pltpu.VMEM((B, 1), jnp.int32),
        ],
        interpret=interpret,
    )(pts)
    return out.reshape(npoint, B, 3).transpose(1, 2, 0)


def _bq_scan(xs, ys, zs, ibufa, ibuf, cx, cy, cz, R2, NCH, N, iota16):
    del ibufa

    def group_body(c, p):
        if True:
            xv = xs[pl.ds(c * 16, 16)]
            yv = ys[pl.ds(c * 16, 16)]
            zv = zs[pl.ds(c * 16, 16)]
            dx = xv - cx
            dy = yv - cy
            dz = zv - cz
            d = (dx * dx + dy * dy) + dz * dz
            m = d <= R2
            incl = plsc.cumsum(m.astype(jnp.int32))
            tgt = jnp.where(m, p + (incl - 1), N + iota16)
            plsc.store_scatter(ibuf, [tgt], iota16 + c * 16)
            p = p + incl[15]
        return p

    return lax.fori_loop(0, NCH, group_body, jnp.int32(0))


def _bq_gather_xyz(soa, ctrs, radius):
    B, _, N = soa.shape
    S = ctrs.shape[2]
    KN = 64
    WPS = 32 // B
    SCH = S // WPS
    NCH = N // 16
    R2 = np.float32(np.float64(radius) * np.float64(radius))
    mesh = plsc.VectorSubcoreMesh(core_axis_name="c", subcore_axis_name="s")

    @functools.partial(
        pl.kernel,
        out_type=jax.ShapeDtypeStruct((B * S * KN * 3,), jnp.float32),
        mesh=mesh,
        scratch_types=[
            pltpu.VMEM((N,), jnp.float32),
            pltpu.VMEM((N,), jnp.float32),
            pltpu.VMEM((N,), jnp.float32),
            pltpu.VMEM((SCH + 16,), jnp.float32),
            pltpu.VMEM((SCH + 16,), jnp.float32),
            pltpu.VMEM((SCH + 16,), jnp.float32),
            pltpu.VMEM((N + 32,), jnp.int32),
            pltpu.VMEM((N + 32,), jnp.int32),
            pltpu.VMEM((SCH * KN * 3,), jnp.float32),
        ],
        compiler_params=pltpu.CompilerParams(needs_layout_passes=False),
    )
    def bq(soa_hbm, ctr_hbm, out_hbm,
           xs, ys, zs, cxv, cyv, czv, ibufa, ibuf, outv):
        b = lax.axis_index("s")
        h = lax.axis_index("c")
        pltpu.sync_copy(soa_hbm.at[pl.ds((b * 3 + 0) * N, N)], xs)
        pltpu.sync_copy(soa_hbm.at[pl.ds((b * 3 + 1) * N, N)], ys)
        pltpu.sync_copy(soa_hbm.at[pl.ds((b * 3 + 2) * N, N)], zs)
        pltpu.sync_copy(ctr_hbm.at[pl.ds((b * 3 + 0) * S + h * SCH, SCH)],
                        cxv.at[pl.ds(0, SCH)])
        pltpu.sync_copy(ctr_hbm.at[pl.ds((b * 3 + 1) * S + h * SCH, SCH)],
                        cyv.at[pl.ds(0, SCH)])
        pltpu.sync_copy(ctr_hbm.at[pl.ds((b * 3 + 2) * S + h * SCH, SCH)],
                        czv.at[pl.ds(0, SCH)])
        iota16 = lax.iota(jnp.int32, 16)

        def center_body(sl, _):
            cx = cxv[pl.ds(sl, 16)][0]
            cy = cyv[pl.ds(sl, 16)][0]
            cz = czv[pl.ds(sl, 16)][0]
            p = _bq_scan(xs, ys, zs, ibufa, ibuf, cx, cy, cz, R2, NCH, N, iota16)
            count = jnp.minimum(p, KN)
            first = ibuf[pl.ds(0, 16)][0]
            for j4 in range(KN // 16):
                iv = ibuf[pl.ds(j4 * 16, 16)]
                pos = iota16 + j4 * 16
                iv = jnp.where(pos < count, iv, first)
                tgt = sl * KN + pos
                plsc.store_scatter(outv, [tgt],
                                   plsc.load_gather(xs, [iv]) - cx)
                plsc.store_scatter(outv, [tgt + SCH * KN],
                                   plsc.load_gather(ys, [iv]) - cy)
                plsc.store_scatter(outv, [tgt + 2 * SCH * KN],
                                   plsc.load_gather(zs, [iv]) - cz)
            return 0

        lax.fori_loop(0, SCH, center_body, 0)
        base = (b * S + h * SCH) * KN
        for pidx in range(3):
            pltpu.sync_copy(
                outv.at[pl.ds(pidx * SCH * KN, SCH * KN)],
                out_hbm.at[pl.ds(pidx * B * S * KN + base, SCH * KN)])

    return bq(soa.reshape(B * 3 * N), ctrs.reshape(B * 3 * S))


def _bq_gather_feat(soa, ctrs, tab, radius):
    B, _, N = soa.shape
    S = ctrs.shape[2]
    CR = tab.shape[1]
    KN = 64
    WPS = 32 // B
    SCH = S // WPS
    NCH = N // 16
    R2 = np.float32(np.float64(radius) * np.float64(radius))
    mesh = plsc.VectorSubcoreMesh(core_axis_name="c", subcore_axis_name="s")

    @functools.partial(
        pl.kernel,
        out_type=jax.ShapeDtypeStruct((B * S * KN, CR), jnp.float32),
        mesh=mesh,
        scratch_types=[
            pltpu.VMEM((N,), jnp.float32),
            pltpu.VMEM((N,), jnp.float32),
            pltpu.VMEM((N,), jnp.float32),
            pltpu.VMEM((SCH + 16,), jnp.float32),
            pltpu.VMEM((SCH + 16,), jnp.float32),
            pltpu.VMEM((SCH + 16,), jnp.float32),
            pltpu.VMEM((N + 32,), jnp.int32),
            pltpu.VMEM((N + 32,), jnp.int32),
            pltpu.VMEM((KN,), jnp.int32),
            pltpu.VMEM((KN,), jnp.int32),
            pltpu.VMEM((KN, CR), jnp.float32),
            pltpu.VMEM((KN, CR), jnp.float32),
            pltpu.SemaphoreType.DMA,
            pltpu.SemaphoreType.DMA,
            pltpu.SemaphoreType.DMA,
            pltpu.SemaphoreType.DMA,
        ],
        compiler_params=pltpu.CompilerParams(needs_layout_passes=False),
    )
    def bq(soa_hbm, ctr_hbm, tab_hbm, out_hbm, xs, ys, zs, cxv, cyv, czv,
           ibufa, ibuf, idxb0, idxb1, rowb0, rowb1, gsem0, gsem1,
           osem0, osem1):
        b = lax.axis_index("s")
        h = lax.axis_index("c")
        pltpu.sync_copy(soa_hbm.at[pl.ds((b * 3 + 0) * N, N)], xs)
        pltpu.sync_copy(soa_hbm.at[pl.ds((b * 3 + 1) * N, N)], ys)
        pltpu.sync_copy(soa_hbm.at[pl.ds((b * 3 + 2) * N, N)], zs)
        pltpu.sync_copy(ctr_hbm.at[pl.ds((b * 3 + 0) * S + h * SCH, SCH)],
                        cxv.at[pl.ds(0, SCH)])
        pltpu.sync_copy(ctr_hbm.at[pl.ds((b * 3 + 1) * S + h * SCH, SCH)],
                        cyv.at[pl.ds(0, SCH)])
        pltpu.sync_copy(ctr_hbm.at[pl.ds((b * 3 + 2) * S + h * SCH, SCH)],
                        czv.at[pl.ds(0, SCH)])
        iota16 = lax.iota(jnp.int32, 16)
        idxbs = [idxb0, idxb1]
        rowbs = [rowb0, rowb1]
        gsems = [gsem0, gsem1]
        osems = [osem0, osem1]
        base = b * S + h * SCH

        def out_slice(sl):
            return out_hbm.at[pl.ds((base + sl) * KN, KN)]

        def pair_body(k, _):
            for u in range(2):
                sl = k * 2 + u
                cx = cxv[pl.ds(sl, 16)][0]
                cy = cyv[pl.ds(sl, 16)][0]
                cz = czv[pl.ds(sl, 16)][0]
                p = _bq_scan(xs, ys, zs, ibufa, ibuf, cx, cy, cz, R2, NCH, N,
                             iota16)

                @pl.when(sl >= 2)
                def _wait_out():
                    pltpu.make_async_copy(rowbs[u], out_slice(sl - 2),
                                          osems[u]).wait()

                count = jnp.minimum(p, KN)
                first = ibuf[pl.ds(0, 16)][0]
                for j4 in range(KN // 16):
                    iv = ibuf[pl.ds(j4 * 16, 16)]
                    pos = iota16 + j4 * 16
                    iv = jnp.where(pos < count, iv, first)
                    idxbs[u][pl.ds(j4 * 16, 16)] = iv + b * N

                pltpu.async_copy(tab_hbm.at[idxbs[u]], rowbs[u], gsems[u])

                @pl.when(sl >= 1)
                def _drain_prev():
                    v = 1 - u
                    pltpu.make_async_copy(tab_hbm.at[idxbs[v]], rowbs[v],
                                          gsems[v]).wait()
                    pltpu.async_copy(rowbs[v], out_slice(sl - 1), osems[v])
            return 0

        lax.fori_loop(0, SCH // 2, pair_body, 0)
        ul = (SCH - 1) % 2
        pltpu.make_async_copy(tab_hbm.at[idxbs[ul]], rowbs[ul],
                              gsems[ul]).wait()
        pltpu.async_copy(rowbs[ul], out_slice(SCH - 1), osems[ul])
        pltpu.make_async_copy(rowbs[1 - ul], out_slice(SCH - 2),
                              osems[1 - ul]).wait()
        pltpu.make_async_copy(rowbs[ul], out_slice(SCH - 1), osems[ul]).wait()

    return bq(soa.reshape(B * 3 * N), ctrs.reshape(B * 3 * S), tab)


def _mlp_max_t(gT, wbs, group=64, tm=8192, interpret=False):
    Cin, R = gT.shape
    G = tm // group
    nlayer = len(wbs)
    Cout = wbs[-1][0].shape[1]

    def kern(g_ref, *refs):
        o_ref = refs[-1]
        h = g_ref[...]
        for i in range(nlayer):
            wT = refs[2 * i][...]
            bc = refs[2 * i + 1][...]
            h = jnp.maximum(
                jnp.dot(wT, h, preferred_element_type=jnp.float32) + bc, 0.0)
        t = h.T
        o_ref[...] = jnp.max(t.reshape(G, group, Cout), axis=1)

    in_specs = [pl.BlockSpec((Cin, tm), lambda i: (0, i))]
    args = [gT]
    for W, bvec in wbs:
        in_specs.append(pl.BlockSpec(W.shape[::-1], lambda i: (0, 0)))
        in_specs.append(pl.BlockSpec((W.shape[1], 1), lambda i: (0, 0)))
        args.append(W.T)
        args.append(bvec.reshape(-1, 1))

    return pl.pallas_call(
        kern,
        grid=(R // tm,),
        in_specs=in_specs,
        out_specs=pl.BlockSpec((G, Cout), lambda i: (i, 0)),
        out_shape=jax.ShapeDtypeStruct((R // group, Cout), jnp.float32),
        interpret=interpret,
    )(*args)


def _proj2(ctr1_soa, f1, ctr2_soa, W1, b1, interpret=False):
    _, R1 = ctr1_soa.shape
    _, R2 = ctr2_soa.shape
    C = W1.shape[1]
    W1xT = W1[0:3].T
    W1f = W1[3:]

    def kern(c1_ref, f1_ref, c2_ref, wxT_ref, wf_ref, b_ref, a_ref, d_ref):
        wxT = wxT_ref[...]
        a_ref[...] = (
            jnp.dot(wxT, c1_ref[...], preferred_element_type=jnp.float32).T
            + jnp.dot(f1_ref[...], wf_ref[...],
                      preferred_element_type=jnp.float32)
            + b_ref[...])
        d_ref[...] = jnp.dot(
            wxT, c2_ref[...], preferred_element_type=jnp.float32).T

    return pl.pallas_call(
        kern,
        out_shape=(jax.ShapeDtypeStruct((R1, C), jnp.float32),
                   jax.ShapeDtypeStruct((R2, C), jnp.float32)),
        interpret=interpret,
    )(ctr1_soa, f1, ctr2_soa, W1xT, W1f, b1.reshape(1, -1))


def _mlp_max_sub(gA, D, wbs, group=64, tm=8192, interpret=False):
    R, C = gA.shape
    G = tm // group
    Cout = wbs[-1][0].shape[1]
    nlayer = len(wbs)

    def kern(g_ref, d_ref, *refs):
        o_ref = refs[-1]
        h = g_ref[...]
        d = d_ref[...]
        h = jnp.maximum(
            (h.reshape(G, group, C) - d[:, None, :]).reshape(tm, C), 0.0)
        for i in range(nlayer):
            W = refs[2 * i][...]
            bvec = refs[2 * i + 1][...]
            h = jnp.maximum(
                jnp.dot(h, W, preferred_element_type=jnp.float32) + bvec, 0.0)
        o_ref[...] = jnp.max(h.reshape(G, group, h.shape[-1]), axis=1)

    in_specs = [
        pl.BlockSpec((tm, C), lambda i: (i, 0)),
        pl.BlockSpec((G, C), lambda i: (i, 0)),
    ]
    args = [gA, D]
    for W, bvec in wbs:
        in_specs.append(pl.BlockSpec(W.shape, lambda i: (0, 0)))
        in_specs.append(pl.BlockSpec((1, W.shape[1]), lambda i: (0, 0)))
        args.append(W)
        args.append(bvec.reshape(1, -1))

    return pl.pallas_call(
        kern,
        grid=(R // tm,),
        in_specs=in_specs,
        out_specs=pl.BlockSpec((G, Cout), lambda i: (i, 0)),
        out_shape=jax.ShapeDtypeStruct((R // group, Cout), jnp.float32),
        interpret=interpret,
    )(*args)


def _mlp_max(g, cpad, wbs, group=64, tm=8192, interpret=False):
    R, Cin = g.shape
    G = tm // group
    grid = R // tm
    Cout = wbs[-1][0].shape[1]
    nlayer = len(wbs)

    def kern(g_ref, c_ref, *refs):
        o_ref = refs[-1]
        h = g_ref[...]
        ctr = c_ref[...]
        h = (h.reshape(G, group, Cin) - ctr[:, None, :]).reshape(tm, Cin)
        for i in range(nlayer):
            W = refs[2 * i][...]
            bvec = refs[2 * i + 1][...]
            h = jnp.maximum(
                jnp.dot(h, W, preferred_element_type=jnp.float32) + bvec, 0.0)
        o_ref[...] = jnp.max(h.reshape(G, group, h.shape[-1]), axis=1)

    in_specs = [
        pl.BlockSpec((tm, Cin), lambda i: (i, 0)),
        pl.BlockSpec((G, Cin), lambda i: (i, 0)),
    ]
    args = [g, cpad]
    for W, bvec in wbs:
        in_specs.append(pl.BlockSpec(W.shape, lambda i: (0, 0)))
        in_specs.append(pl.BlockSpec((1, W.shape[1]), lambda i: (0, 0)))
        args.append(W)
        args.append(bvec.reshape(1, -1))

    return pl.pallas_call(
        kern,
        grid=(grid,),
        in_specs=in_specs,
        out_specs=pl.BlockSpec((G, Cout), lambda i: (i, 0)),
        out_shape=jax.ShapeDtypeStruct((R // group, Cout), jnp.float32),
        interpret=interpret,
    )(*args)


def _head(rows, B, wbs, interpret=False):
    R, Cin = rows.shape
    S = R // B
    nlayer = len(wbs)

    def kern(r_ref, *refs):
        o_ref = refs[-1]
        h = r_ref[...]
        for i in range(3):
            W = refs[2 * i][...]
            bvec = refs[2 * i + 1][...]
            h = jnp.maximum(
                jnp.dot(h, W, preferred_element_type=jnp.float32) + bvec, 0.0)
        x = jnp.max(h.reshape(B, S, h.shape[-1]), axis=1)
        for i in range(3, nlayer - 1):
            W = refs[2 * i][...]
            bvec = refs[2 * i + 1][...]
            x = jnp.maximum(
                jnp.dot(x, W, preferred_element_type=jnp.float32) + bvec, 0.0)
        W = refs[2 * (nlayer - 1)][...]
        bvec = refs[2 * (nlayer - 1) + 1][...]
        o_ref[...] = jnp.dot(x, W, preferred_element_type=jnp.float32) + bvec

    args = [rows]
    for W, bvec in wbs:
        args.append(W)
        args.append(bvec.reshape(1, -1))

    return pl.pallas_call(
        kern,
        out_shape=jax.ShapeDtypeStruct((B, wbs[-1][0].shape[1]), jnp.float32),
        interpret=interpret,
    )(*args)


def _pad_rows(W, rows_to):
    return jnp.concatenate(
        [W, jnp.zeros((rows_to - W.shape[0], W.shape[1]), W.dtype)], axis=0)


def kernel(points, params):
    sa1, sa2, sa3, fc, lin = params
    B, _, N1 = points.shape
    S1, S2, KN = 512, 128, 64

    ctr1 = _fps(points, S1)
    g1 = _bq_gather_xyz(points, ctr1, 0.2).reshape(3, B * S1 * KN)
    ctr1_rows = ctr1.transpose(0, 2, 1).reshape(B * S1, 3)
    f1 = _mlp_max_t(g1, list(sa1), group=KN)

    ctr2 = _fps(ctr1, S2)
    ctr1_soa = ctr1.transpose(1, 0, 2).reshape(3, B * S1)
    ctr2_soa = ctr2.transpose(1, 0, 2).reshape(3, B * S2)
    A2, D2 = _proj2(ctr1_soa, f1, ctr2_soa, sa2[0][0], sa2[0][1])
    gA = _bq_gather_feat(ctr1, ctr2, A2, 0.4)
    ctr2_rows = ctr2.transpose(0, 2, 1).reshape(B * S2, 3)
    f2 = _mlp_max_sub(gA, D2, list(sa2[1:]), group=KN)

    rows3 = jnp.concatenate([ctr2_rows, f2], axis=1)
    wbs3 = list(sa3) + list(fc) + [lin]
    return _head(rows3, B, wbs3)
